# Initial kernel scaffold; baseline (speedup 1.0000x reference)
import jax
import jax.numpy as jnp
from jax.experimental import pallas as pl


def kernel(row_emb, col_emb, cost_mat, W_q, b_q, W_k, b_k, W_v, b_v, W_o, b_o, beta, ms_W1, ms_b1, ms_W2, ms_b2):
    raise NotImplementedError("write your pallas kernel here")


# speedup vs baseline: 1.5254x; 1.5254x over previous
import functools

import jax
import jax.numpy as jnp
from jax.experimental import pallas as pl

BS, ROW, COL, D, H = 4, 512, 512, 128, 8
HD = D // H
MSH = 32
TANH_CLIP = 10.0


def _body(row_ref, col_ref, cost_ref, wq_ref, bq_ref, wk_ref, bk_ref,
          wv_ref, bv_ref, wot_ref, bo_ref, beta_ref, w1_ref, b1_ref,
          w2_ref, b2_ref, out_ref):
    h = pl.program_id(1)
    row = row_ref[0]
    col = col_ref[0]
    cost = cost_ref[0]
    dn_nt = (((1,), (1,)), ((), ()))
    dn_nn = (((1,), (0,)), ((), ()))
    f32 = jnp.float32
    q = jax.lax.dot_general(row, wq_ref[...], dn_nt,
                            preferred_element_type=f32, precision=jax.lax.Precision.HIGHEST) + bq_ref[0]
    k = jax.lax.dot_general(col, wk_ref[...], dn_nt,
                            preferred_element_type=f32, precision=jax.lax.Precision.HIGHEST) + bk_ref[0]
    v = jax.lax.dot_general(col, wv_ref[...], dn_nt,
                            preferred_element_type=f32, precision=jax.lax.Precision.HIGHEST) + bv_ref[0]
    logits = jax.lax.dot_general(q, k, dn_nt, preferred_element_type=f32, precision=jax.lax.Precision.HIGHEST)
    logits = logits * (1.0 / (HD ** 0.5))

    a = w1_ref[0, 0]
    c = w1_ref[0, 1]
    b1 = b1_ref[0, 0]
    w2 = w2_ref[0, 0]
    acc = jnp.full((ROW, COL), b2_ref[0, 0, 0], f32)
    for j in range(MSH):
        hj = jnp.maximum(logits * a[j] + (cost * c[j] + b1[j]), 0.0)
        acc = acc + hj * w2[j]

    scores = jnp.where(cost > 0.0, jnp.tanh(acc) * TANH_CLIP,
                       beta_ref[0, 0, 0])
    m = jnp.max(scores, axis=1, keepdims=True)
    e = jnp.exp(scores - m)
    s = jnp.sum(e, axis=1, keepdims=True)
    wts = e * (1.0 / s)
    head = jax.lax.dot_general(wts, v, dn_nn, preferred_element_type=f32, precision=jax.lax.Precision.HIGHEST)
    contrib = jax.lax.dot_general(head, wot_ref[...], dn_nn,
                                  preferred_element_type=f32, precision=jax.lax.Precision.HIGHEST)

    @pl.when(h == 0)
    def _init():
        out_ref[0] = contrib + bo_ref[0]

    @pl.when(h != 0)
    def _accum():
        out_ref[0] = out_ref[0] + contrib


@functools.partial(jax.jit, static_argnames=("interpret",))
def kernel(row_emb, col_emb, cost_mat, W_q, b_q, W_k, b_k, W_v, b_v,
           W_o, b_o, beta, ms_W1, ms_b1, ms_W2, ms_b2, interpret=False):
    grid = (BS, H)
    w_spec = pl.BlockSpec((HD, D), lambda b, h: (h, 0))
    bias_spec = pl.BlockSpec((1, 1, HD), lambda b, h: (h, 0, 0))
    hs_spec = pl.BlockSpec((1, 1, MSH), lambda b, h: (h, 0, 0))
    scal_spec = pl.BlockSpec((1, 1, 1), lambda b, h: (h, 0, 0))
    out = pl.pallas_call(
        _body,
        grid=grid,
        in_specs=[
            pl.BlockSpec((1, ROW, D), lambda b, h: (b, 0, 0)),
            pl.BlockSpec((1, COL, D), lambda b, h: (b, 0, 0)),
            pl.BlockSpec((1, ROW, COL), lambda b, h: (b, 0, 0)),
            w_spec,
            bias_spec,
            w_spec,
            bias_spec,
            w_spec,
            bias_spec,
            w_spec,
            pl.BlockSpec((1, D), lambda b, h: (0, 0)),
            scal_spec,
            pl.BlockSpec((1, 2, MSH), lambda b, h: (h, 0, 0)),
            hs_spec,
            hs_spec,
            scal_spec,
        ],
        out_specs=pl.BlockSpec((1, ROW, D), lambda b, h: (b, 0, 0)),
        out_shape=jax.ShapeDtypeStruct((BS, ROW, D), jnp.float32),
        interpret=interpret,
    )(
        row_emb, col_emb, cost_mat,
        W_q, b_q.reshape(H, 1, HD),
        W_k, b_k.reshape(H, 1, HD),
        W_v, b_v.reshape(H, 1, HD),
        W_o.T, b_o.reshape(1, D),
        beta.reshape(H, 1, 1),
        ms_W1, ms_b1.reshape(H, 1, MSH),
        ms_W2.reshape(H, 1, MSH), ms_b2.reshape(H, 1, 1),
    )
    return out


# speedup vs baseline: 1.5559x; 1.0200x over previous
import functools

import jax
import jax.numpy as jnp
from jax.experimental import pallas as pl

BS, ROW, COL, D, H = 4, 512, 512, 128, 8
HD = D // H
MSH = 32
TANH_CLIP = 10.0


def _body(row_ref, col_ref, cost_ref, wq_ref, bq_ref, wk_ref, bk_ref,
          wv_ref, bv_ref, wot_ref, bo_ref, beta_ref, w1_ref, b1_ref,
          w2_ref, b2_ref, out_ref):
    h = pl.program_id(1)
    row = row_ref[0]
    col = col_ref[0]
    cost = cost_ref[0]
    dn_nt = (((1,), (1,)), ((), ()))
    dn_nn = (((1,), (0,)), ((), ()))
    f32 = jnp.float32
    hi = jax.lax.Precision.HIGHEST
    q = jax.lax.dot_general(row, wq_ref[...], dn_nt,
                            preferred_element_type=f32, precision=hi) + bq_ref[0]
    k = jax.lax.dot_general(col, wk_ref[...], dn_nt,
                            preferred_element_type=f32, precision=hi) + bk_ref[0]
    v = jax.lax.dot_general(col, wv_ref[...], dn_nt,
                            preferred_element_type=f32, precision=hi) + bv_ref[0]
    logits = jax.lax.dot_general(q, k, dn_nt, preferred_element_type=f32,
                                 precision=hi)

    a = w1_ref[0, 0]
    c = w1_ref[0, 1]
    b1 = b1_ref[0, 0]
    w2 = w2_ref[0, 0]
    acc0 = jnp.full((ROW, COL), b2_ref[0, 0, 0], f32)
    acc1 = jnp.zeros((ROW, COL), f32)
    for j in range(0, MSH, 2):
        h0 = jnp.maximum(logits * a[j] + (cost * c[j] + b1[j]), 0.0)
        h1 = jnp.maximum(logits * a[j + 1] + (cost * c[j + 1] + b1[j + 1]), 0.0)
        acc0 = acc0 + h0 * w2[j]
        acc1 = acc1 + h1 * w2[j + 1]
    acc = acc0 + acc1

    log2e = 1.4426950408889634
    t = jnp.tanh(acc)
    e = jnp.where(cost > 0.0, jnp.exp2(t * (TANH_CLIP * log2e)),
                  jnp.exp2(beta_ref[0, 0, 0] * log2e))
    s = jnp.sum(e, axis=1, keepdims=True)
    head = jax.lax.dot_general(e, v, dn_nn, preferred_element_type=f32,
                               precision=hi)
    head = head * (1.0 / s)
    contrib = jax.lax.dot_general(head, wot_ref[...], dn_nn,
                                  preferred_element_type=f32, precision=hi)

    @pl.when(h == 0)
    def _init():
        out_ref[0] = contrib + bo_ref[0]

    @pl.when(h != 0)
    def _accum():
        out_ref[0] = out_ref[0] + contrib


@functools.partial(jax.jit, static_argnames=("interpret",))
def kernel(row_emb, col_emb, cost_mat, W_q, b_q, W_k, b_k, W_v, b_v,
           W_o, b_o, beta, ms_W1, ms_b1, ms_W2, ms_b2, interpret=False):
    grid = (BS, H)
    w_spec = pl.BlockSpec((HD, D), lambda b, h: (h, 0))
    bias_spec = pl.BlockSpec((1, 1, HD), lambda b, h: (h, 0, 0))
    hs_spec = pl.BlockSpec((1, 1, MSH), lambda b, h: (h, 0, 0))
    scal_spec = pl.BlockSpec((1, 1, 1), lambda b, h: (h, 0, 0))
    out = pl.pallas_call(
        _body,
        grid=grid,
        in_specs=[
            pl.BlockSpec((1, ROW, D), lambda b, h: (b, 0, 0)),
            pl.BlockSpec((1, COL, D), lambda b, h: (b, 0, 0)),
            pl.BlockSpec((1, ROW, COL), lambda b, h: (b, 0, 0)),
            w_spec,
            bias_spec,
            w_spec,
            bias_spec,
            w_spec,
            bias_spec,
            w_spec,
            pl.BlockSpec((1, D), lambda b, h: (0, 0)),
            scal_spec,
            pl.BlockSpec((1, 2, MSH), lambda b, h: (h, 0, 0)),
            hs_spec,
            hs_spec,
            scal_spec,
        ],
        out_specs=pl.BlockSpec((1, ROW, D), lambda b, h: (b, 0, 0)),
        out_shape=jax.ShapeDtypeStruct((BS, ROW, D), jnp.float32),
        interpret=interpret,
    )(
        row_emb, col_emb, cost_mat,
        W_q, b_q.reshape(H, 1, HD),
        W_k, b_k.reshape(H, 1, HD),
        W_v, b_v.reshape(H, 1, HD),
        W_o.T, b_o.reshape(1, D),
        beta.reshape(H, 1, 1),
        ms_W1 * jnp.array([1.0 / (HD ** 0.5), 1.0], jnp.float32)[None, :, None],
        ms_b1.reshape(H, 1, MSH),
        ms_W2.reshape(H, 1, MSH), ms_b2.reshape(H, 1, 1),
    )
    return out
